# baseline (speedup 1.0000x reference)
import jax
import jax.numpy as jnp
from jax.experimental import pallas as pl
from jax.experimental.pallas import tpu as pltpu

_BN = 4096
_T = 32
_BL = 1024
_K = _BN // _BL
_NH = 256
_NC = 10
_B = 4
_N = 1024


def _body(x_ref, e_ref, nw_ref, nb_ref, w1_ref, b1_ref, w2_ref, b2_ref,
          o1w_ref, o1b_ref, o2w_ref, o2b_ref, o3w_ref, o3b_ref, out_ref,
          xbf_s, ebf_s, y1_s, y2_s, agg_s):
    i = pl.program_id(0)
    k = pl.program_id(1)

    @pl.when((i == 0) & (k == 0))
    def _init():
        x = jnp.maximum(
            jnp.dot(x_ref[...], nw_ref[...],
                    preferred_element_type=jnp.float32) + nb_ref[...], 0.0)
        xbf_s[...] = x.astype(jnp.bfloat16)

    @pl.when(i == 1)
    def _agg2_type1_chunk():
        a = jnp.dot(ebf_s[pl.ds(k * _BL, _BL), :].astype(jnp.bfloat16),
                    y1_s[...],
                    preferred_element_type=jnp.float32)
        agg_s[pl.ds(k * _BL, _BL), :] = jnp.maximum(
            jnp.dot(a, w2_ref[1], preferred_element_type=jnp.float32)
            + b2_ref[1], 0.0).astype(jnp.bfloat16)

    eb = e_ref[0].astype(jnp.bfloat16)
    ebf_s[pl.ds(k * _BL, _BL), :] = eb.astype(jnp.float8_e4m3fn)
    z = jnp.dot(eb, xbf_s[...], preferred_element_type=jnp.float32)
    y = jnp.maximum(
        jnp.dot(z, w1_ref[0], preferred_element_type=jnp.float32)
        + b1_ref[0], 0.0)
    yb = y.astype(jnp.bfloat16)

    @pl.when(i == 0)
    def _store_y1():
        y1_s[pl.ds(k * _BL, _BL), :] = yb

    @pl.when(i == 1)
    def _store_y2():
        y2_s[pl.ds(k * _BL, _BL), :] = yb

    @pl.when((i == 1) & (k == _K - 1))
    def _final():
        a = jnp.dot(ebf_s[...].astype(jnp.bfloat16), y2_s[...],
                    preferred_element_type=jnp.float32)
        agg = agg_s[...].astype(jnp.float32) + jnp.maximum(
            jnp.dot(a, w2_ref[2], preferred_element_type=jnp.float32)
            + b2_ref[2], 0.0)
        p = jnp.maximum(
            jnp.dot(agg.astype(jnp.bfloat16),
                    o1w_ref[...].astype(jnp.bfloat16),
                    preferred_element_type=jnp.float32) + o1b_ref[...], 0.0)
        p = jnp.maximum(
            jnp.dot(p.astype(jnp.bfloat16), o2w_ref[...].astype(jnp.bfloat16),
                    preferred_element_type=jnp.float32) + o2b_ref[...], 0.0)
        pm = jnp.mean(p.reshape(_B, _N, _NH), axis=1)
        out_ref[...] = (jnp.dot(pm, o3w_ref[...],
                                preferred_element_type=jnp.float32)
                        + o3b_ref[...])


def kernel(inputs, sparse_edges, node_fc1_w, node_fc1_b, msg_fc1_w, msg_fc1_b,
           msg_fc2_w, msg_fc2_b, out_fc1_w, out_fc1_b, out_fc2_w, out_fc2_b,
           out_fc3_w, out_fc3_b):
    x2d = inputs.reshape(_BN, _T).astype(jnp.bfloat16)
    nw = node_fc1_w.astype(jnp.bfloat16)
    nb = node_fc1_b.reshape(1, _T)
    b1 = msg_fc1_b.reshape(-1, 1, _T)
    b2 = msg_fc2_b.reshape(-1, 1, _T)
    o1b = out_fc1_b.reshape(1, _NH)
    o2b = out_fc2_b.reshape(1, _NH)
    o3b = out_fc3_b.reshape(1, _NC)

    grid = (2, _K)
    in_specs = [
        pl.BlockSpec((_BN, _T), lambda i, k: (0, 0)),
        pl.BlockSpec((1, _BL, _BN), lambda i, k: (i + 1, k, 0)),
        pl.BlockSpec((_T, _T), lambda i, k: (0, 0)),
        pl.BlockSpec((1, _T), lambda i, k: (0, 0)),
        pl.BlockSpec((1, _T, _T), lambda i, k: (i + 1, 0, 0)),
        pl.BlockSpec((1, 1, _T), lambda i, k: (i + 1, 0, 0)),
        pl.BlockSpec((3, _T, _T), lambda i, k: (0, 0, 0)),
        pl.BlockSpec((3, 1, _T), lambda i, k: (0, 0, 0)),
        pl.BlockSpec((_T, _NH), lambda i, k: (0, 0)),
        pl.BlockSpec((1, _NH), lambda i, k: (0, 0)),
        pl.BlockSpec((_NH, _NH), lambda i, k: (0, 0)),
        pl.BlockSpec((1, _NH), lambda i, k: (0, 0)),
        pl.BlockSpec((_NH, _NC), lambda i, k: (0, 0)),
        pl.BlockSpec((1, _NC), lambda i, k: (0, 0)),
    ]
    out_spec = pl.BlockSpec((_B, _NC), lambda i, k: (0, 0))
    scratch_shapes = [
        pltpu.VMEM((_BN, _T), jnp.bfloat16),
        pltpu.VMEM((_BN, _BN), jnp.float8_e4m3fn),
        pltpu.VMEM((_BN, _T), jnp.bfloat16),
        pltpu.VMEM((_BN, _T), jnp.bfloat16),
        pltpu.VMEM((_BN, _T), jnp.bfloat16),
    ]
    params_cls = getattr(pltpu, "CompilerParams", None) or pltpu.TPUCompilerParams
    return pl.pallas_call(
        _body,
        grid=grid,
        in_specs=in_specs,
        out_specs=out_spec,
        out_shape=jax.ShapeDtypeStruct((_B, _NC), jnp.float32),
        scratch_shapes=scratch_shapes,
        compiler_params=params_cls(
            dimension_semantics=("arbitrary", "arbitrary"),
            vmem_limit_bytes=67043328,
        ),
    )(x2d, sparse_edges, nw, nb, msg_fc1_w, b1, msg_fc2_w, b2,
      out_fc1_w, o1b, out_fc2_w, o2b, out_fc3_w, o3b)


# speedup vs baseline: 1.0109x; 1.0109x over previous
import jax
import jax.numpy as jnp
from jax.experimental import pallas as pl
from jax.experimental.pallas import tpu as pltpu

_BN = 4096
_T = 32
_BL = 512
_K = _BN // _BL
_NH = 256
_NC = 10
_B = 4
_N = 1024


def _body(x_ref, e_ref, nw_ref, nb_ref, w1_ref, b1_ref, w2_ref, b2_ref,
          o1w_ref, o1b_ref, o2w_ref, o2b_ref, o3w_ref, o3b_ref, out_ref,
          xbf_s, ebf_s, y1_s, y2_s, agg_s):
    i = pl.program_id(0)
    k = pl.program_id(1)

    @pl.when((i == 0) & (k == 0))
    def _init():
        x = jnp.maximum(
            jnp.dot(x_ref[...], nw_ref[...],
                    preferred_element_type=jnp.float32) + nb_ref[...], 0.0)
        xbf_s[...] = x.astype(jnp.bfloat16)

    @pl.when(i == 1)
    def _agg2_type1_chunk():
        a = jnp.dot(ebf_s[pl.ds(k * _BL, _BL), :].astype(jnp.bfloat16),
                    y1_s[...],
                    preferred_element_type=jnp.float32)
        agg_s[pl.ds(k * _BL, _BL), :] = jnp.maximum(
            jnp.dot(a, w2_ref[1], preferred_element_type=jnp.float32)
            + b2_ref[1], 0.0).astype(jnp.bfloat16)

    eb = e_ref[0].astype(jnp.bfloat16)
    ebf_s[pl.ds(k * _BL, _BL), :] = eb.astype(jnp.float8_e4m3fn)
    z = jnp.dot(eb, xbf_s[...], preferred_element_type=jnp.float32)
    y = jnp.maximum(
        jnp.dot(z, w1_ref[0], preferred_element_type=jnp.float32)
        + b1_ref[0], 0.0)
    yb = y.astype(jnp.bfloat16)

    @pl.when(i == 0)
    def _store_y1():
        y1_s[pl.ds(k * _BL, _BL), :] = yb

    @pl.when(i == 1)
    def _store_y2():
        y2_s[pl.ds(k * _BL, _BL), :] = yb

    @pl.when((i == 1) & (k == _K - 1))
    def _final():
        a = jnp.dot(ebf_s[...].astype(jnp.bfloat16), y2_s[...],
                    preferred_element_type=jnp.float32)
        agg = agg_s[...].astype(jnp.float32) + jnp.maximum(
            jnp.dot(a, w2_ref[2], preferred_element_type=jnp.float32)
            + b2_ref[2], 0.0)
        p = jnp.maximum(
            jnp.dot(agg.astype(jnp.bfloat16),
                    o1w_ref[...].astype(jnp.bfloat16),
                    preferred_element_type=jnp.float32) + o1b_ref[...], 0.0)
        p = jnp.maximum(
            jnp.dot(p.astype(jnp.bfloat16), o2w_ref[...].astype(jnp.bfloat16),
                    preferred_element_type=jnp.float32) + o2b_ref[...], 0.0)
        pm = jnp.mean(p.reshape(_B, _N, _NH), axis=1)
        out_ref[...] = (jnp.dot(pm, o3w_ref[...],
                                preferred_element_type=jnp.float32)
                        + o3b_ref[...])


def kernel(inputs, sparse_edges, node_fc1_w, node_fc1_b, msg_fc1_w, msg_fc1_b,
           msg_fc2_w, msg_fc2_b, out_fc1_w, out_fc1_b, out_fc2_w, out_fc2_b,
           out_fc3_w, out_fc3_b):
    x2d = inputs.reshape(_BN, _T).astype(jnp.bfloat16)
    nw = node_fc1_w.astype(jnp.bfloat16)
    nb = node_fc1_b.reshape(1, _T)
    b1 = msg_fc1_b.reshape(-1, 1, _T)
    b2 = msg_fc2_b.reshape(-1, 1, _T)
    o1b = out_fc1_b.reshape(1, _NH)
    o2b = out_fc2_b.reshape(1, _NH)
    o3b = out_fc3_b.reshape(1, _NC)

    grid = (2, _K)
    in_specs = [
        pl.BlockSpec((_BN, _T), lambda i, k: (0, 0)),
        pl.BlockSpec((1, _BL, _BN), lambda i, k: (i + 1, k, 0)),
        pl.BlockSpec((_T, _T), lambda i, k: (0, 0)),
        pl.BlockSpec((1, _T), lambda i, k: (0, 0)),
        pl.BlockSpec((1, _T, _T), lambda i, k: (i + 1, 0, 0)),
        pl.BlockSpec((1, 1, _T), lambda i, k: (i + 1, 0, 0)),
        pl.BlockSpec((3, _T, _T), lambda i, k: (0, 0, 0)),
        pl.BlockSpec((3, 1, _T), lambda i, k: (0, 0, 0)),
        pl.BlockSpec((_T, _NH), lambda i, k: (0, 0)),
        pl.BlockSpec((1, _NH), lambda i, k: (0, 0)),
        pl.BlockSpec((_NH, _NH), lambda i, k: (0, 0)),
        pl.BlockSpec((1, _NH), lambda i, k: (0, 0)),
        pl.BlockSpec((_NH, _NC), lambda i, k: (0, 0)),
        pl.BlockSpec((1, _NC), lambda i, k: (0, 0)),
    ]
    out_spec = pl.BlockSpec((_B, _NC), lambda i, k: (0, 0))
    scratch_shapes = [
        pltpu.VMEM((_BN, _T), jnp.bfloat16),
        pltpu.VMEM((_BN, _BN), jnp.float8_e4m3fn),
        pltpu.VMEM((_BN, _T), jnp.bfloat16),
        pltpu.VMEM((_BN, _T), jnp.bfloat16),
        pltpu.VMEM((_BN, _T), jnp.bfloat16),
    ]
    params_cls = getattr(pltpu, "CompilerParams", None) or pltpu.TPUCompilerParams
    return pl.pallas_call(
        _body,
        grid=grid,
        in_specs=in_specs,
        out_specs=out_spec,
        out_shape=jax.ShapeDtypeStruct((_B, _NC), jnp.float32),
        scratch_shapes=scratch_shapes,
        compiler_params=params_cls(
            dimension_semantics=("arbitrary", "arbitrary"),
            vmem_limit_bytes=67043328,
        ),
    )(x2d, sparse_edges, nw, nb, msg_fc1_w, b1, msg_fc2_w, b2,
      out_fc1_w, o1b, out_fc2_w, o2b, out_fc3_w, o3b)


# speedup vs baseline: 1.0722x; 1.0606x over previous
import jax
import jax.numpy as jnp
from jax.experimental import pallas as pl
from jax.experimental.pallas import tpu as pltpu

_BN = 4096
_T = 32
_TA = 64
_BL = 512
_K = _BN // _BL
_NH = 256
_NC = 10
_B = 4
_N = 1024


def _colmean_offset(y_ref):
    y = y_ref[...].astype(jnp.float32)
    m = jnp.mean(y, axis=0, keepdims=True)
    d = (y - m).astype(jnp.float8_e4m3fn)
    return m, d


def _body(x_ref, e_ref, nw_ref, nb_ref, w1_ref, b1_ref, w2_ref, b2_ref,
          o1w_ref, o1b_ref, o2w_ref, o2b_ref, o3w_ref, o3b_ref, out_ref,
          xbf_s, ebf_s, y1_s, y2_s, agg_s, rs1_s, rs2_s, d1_s, m1_s):
    i = pl.program_id(0)
    k = pl.program_id(1)

    @pl.when((i == 0) & (k == 0))
    def _init():
        x = jnp.maximum(
            jnp.dot(x_ref[...], nw_ref[...],
                    preferred_element_type=jnp.float32) + nb_ref[...], 0.0)
        aug = jnp.concatenate(
            [x.astype(jnp.bfloat16),
             jnp.ones((_BN, 1), jnp.bfloat16),
             jnp.zeros((_BN, _TA - _T - 1), jnp.bfloat16)], axis=1)
        xbf_s[...] = aug

    @pl.when((i == 1) & (k == 0))
    def _offset_y1():
        m1, d1 = _colmean_offset(y1_s)
        m1_s[...] = m1
        d1_s[...] = d1

    @pl.when(i == 1)
    def _agg2_type1_chunk():
        a = (jnp.dot(ebf_s[pl.ds(k * _BL, _BL), :], d1_s[...],
                     preferred_element_type=jnp.float32)
             + rs1_s[pl.ds(k * _BL, _BL), :] * m1_s[...])
        agg_s[pl.ds(k * _BL, _BL), :] = jnp.maximum(
            jnp.dot(a, w2_ref[1], preferred_element_type=jnp.float32)
            + b2_ref[1], 0.0).astype(jnp.bfloat16)

    eb = e_ref[0].astype(jnp.bfloat16)
    ebf_s[pl.ds(k * _BL, _BL), :] = eb.astype(jnp.float8_e4m3fn)
    za = jnp.dot(eb, xbf_s[...], preferred_element_type=jnp.float32)
    y = jnp.maximum(
        jnp.dot(za[:, 0:_T], w1_ref[0], preferred_element_type=jnp.float32)
        + b1_ref[0], 0.0)
    yb = y.astype(jnp.bfloat16)
    rs = za[:, _T:_T + 1]

    @pl.when(i == 0)
    def _store_y1():
        y1_s[pl.ds(k * _BL, _BL), :] = yb
        rs1_s[pl.ds(k * _BL, _BL), :] = rs

    @pl.when(i == 1)
    def _store_y2():
        y2_s[pl.ds(k * _BL, _BL), :] = yb
        rs2_s[pl.ds(k * _BL, _BL), :] = rs

    @pl.when((i == 1) & (k == _K - 1))
    def _final():
        m2, d2 = _colmean_offset(y2_s)
        a = (jnp.dot(ebf_s[...], d2, preferred_element_type=jnp.float32)
             + rs2_s[...] * m2)
        agg = agg_s[...].astype(jnp.float32) + jnp.maximum(
            jnp.dot(a, w2_ref[2], preferred_element_type=jnp.float32)
            + b2_ref[2], 0.0)
        p = jnp.maximum(
            jnp.dot(agg.astype(jnp.bfloat16),
                    o1w_ref[...].astype(jnp.bfloat16),
                    preferred_element_type=jnp.float32) + o1b_ref[...], 0.0)
        p = jnp.maximum(
            jnp.dot(p.astype(jnp.bfloat16), o2w_ref[...].astype(jnp.bfloat16),
                    preferred_element_type=jnp.float32) + o2b_ref[...], 0.0)
        pm = jnp.mean(p.reshape(_B, _N, _NH), axis=1)
        out_ref[...] = (jnp.dot(pm, o3w_ref[...],
                                preferred_element_type=jnp.float32)
                        + o3b_ref[...])


def kernel(inputs, sparse_edges, node_fc1_w, node_fc1_b, msg_fc1_w, msg_fc1_b,
           msg_fc2_w, msg_fc2_b, out_fc1_w, out_fc1_b, out_fc2_w, out_fc2_b,
           out_fc3_w, out_fc3_b):
    x2d = inputs.reshape(_BN, _T).astype(jnp.bfloat16)
    nw = node_fc1_w.astype(jnp.bfloat16)
    nb = node_fc1_b.reshape(1, _T)
    b1 = msg_fc1_b.reshape(-1, 1, _T)
    b2 = msg_fc2_b.reshape(-1, 1, _T)
    o1b = out_fc1_b.reshape(1, _NH)
    o2b = out_fc2_b.reshape(1, _NH)
    o3b = out_fc3_b.reshape(1, _NC)

    grid = (2, _K)
    in_specs = [
        pl.BlockSpec((_BN, _T), lambda i, k: (0, 0)),
        pl.BlockSpec((1, _BL, _BN), lambda i, k: (i + 1, k, 0)),
        pl.BlockSpec((_T, _T), lambda i, k: (0, 0)),
        pl.BlockSpec((1, _T), lambda i, k: (0, 0)),
        pl.BlockSpec((1, _T, _T), lambda i, k: (i + 1, 0, 0)),
        pl.BlockSpec((1, 1, _T), lambda i, k: (i + 1, 0, 0)),
        pl.BlockSpec((3, _T, _T), lambda i, k: (0, 0, 0)),
        pl.BlockSpec((3, 1, _T), lambda i, k: (0, 0, 0)),
        pl.BlockSpec((_T, _NH), lambda i, k: (0, 0)),
        pl.BlockSpec((1, _NH), lambda i, k: (0, 0)),
        pl.BlockSpec((_NH, _NH), lambda i, k: (0, 0)),
        pl.BlockSpec((1, _NH), lambda i, k: (0, 0)),
        pl.BlockSpec((_NH, _NC), lambda i, k: (0, 0)),
        pl.BlockSpec((1, _NC), lambda i, k: (0, 0)),
    ]
    out_spec = pl.BlockSpec((_B, _NC), lambda i, k: (0, 0))
    scratch_shapes = [
        pltpu.VMEM((_BN, _TA), jnp.bfloat16),
        pltpu.VMEM((_BN, _BN), jnp.float8_e4m3fn),
        pltpu.VMEM((_BN, _T), jnp.bfloat16),
        pltpu.VMEM((_BN, _T), jnp.bfloat16),
        pltpu.VMEM((_BN, _T), jnp.bfloat16),
        pltpu.VMEM((_BN, 1), jnp.float32),
        pltpu.VMEM((_BN, 1), jnp.float32),
        pltpu.VMEM((_BN, _T), jnp.float8_e4m3fn),
        pltpu.VMEM((1, _T), jnp.float32),
    ]
    params_cls = getattr(pltpu, "CompilerParams", None) or pltpu.TPUCompilerParams
    return pl.pallas_call(
        _body,
        grid=grid,
        in_specs=in_specs,
        out_specs=out_spec,
        out_shape=jax.ShapeDtypeStruct((_B, _NC), jnp.float32),
        scratch_shapes=scratch_shapes,
        compiler_params=params_cls(
            dimension_semantics=("arbitrary", "arbitrary"),
            vmem_limit_bytes=67043328,
        ),
    )(x2d, sparse_edges, nw, nb, msg_fc1_w, b1, msg_fc2_w, b2,
      out_fc1_w, o1b, out_fc2_w, o2b, out_fc3_w, o3b)
